# baseline (speedup 1.0000x reference)
import functools

import jax
import jax.numpy as jnp
from jax import lax
from jax.experimental import pallas as pl
from jax.experimental.pallas import tpu as pltpu
from jax.experimental.pallas import tpu_sc as plsc

NC, NS, L = 2, 16, 16
NW = NC * NS
B = 4096
L1 = 20
D = 64
KD = D // L
BPW = B // NW
C = 16
NCH = BPW // C
RPC = C * L1
GATHER_SPLIT = (128, 128, 64)


def _body(xi, yi, xf, yf, human, virus, out,
          idx0, idx1, yidx0, yidx1, rows0, rows1, xv0, xv1,
          vrow0, vrow1, yv0, yv1, outv,
          semi0, semi1, semm0, semm1):
    idx_v = (idx0, idx1)
    yidx_v = (yidx0, yidx1)
    rows_v = (rows0, rows1)
    x_v = (xv0, xv1)
    vrow_v = (vrow0, vrow1)
    y_v = (yv0, yv1)
    sem_idx = (semi0, semi1)
    sem_main = (semm0, semm1)

    wid = lax.axis_index("s") * NC + lax.axis_index("c")
    rbase0 = wid * (BPW * L1)
    bbase0 = wid * BPW

    pend_idx = {}
    pend_main = {}

    def prep_idx(ch):
        s = ch % 2
        rb = rbase0 + ch * RPC
        bb = bbase0 + ch * C
        descs = [
            pltpu.make_async_copy(xi.at[pl.ds(rb, RPC)], idx_v[s], sem_idx[s]),
            pltpu.make_async_copy(yi.at[pl.ds(bb, C)], yidx_v[s], sem_idx[s]),
        ]
        for d_ in descs:
            d_.start()
        pend_idx[s] = descs

    def prep_main(ch):
        s = ch % 2
        for d_ in pend_idx.pop(s):
            d_.wait()
        rb = rbase0 + ch * RPC
        bb = bbase0 + ch * C
        descs = []
        off = 0
        for n in GATHER_SPLIT:
            descs.append(pltpu.make_async_copy(
                human.at[idx_v[s].at[pl.ds(off, n)]],
                rows_v[s].at[pl.ds(off, n)], sem_main[s]))
            off += n
        descs.append(pltpu.make_async_copy(
            virus.at[yidx_v[s]], vrow_v[s], sem_main[s]))
        descs.append(pltpu.make_async_copy(
            xf.at[pl.ds(rb, RPC)], x_v[s], sem_main[s]))
        descs.append(pltpu.make_async_copy(
            yf.at[pl.ds(bb, C)], y_v[s], sem_main[s]))
        for d_ in descs:
            d_.start()
        pend_main[s] = descs

    lane_iota = lax.iota(jnp.int32, L)

    def compute(ch):
        s = ch % 2
        for d_ in pend_main.pop(s):
            d_.wait()
        rows = rows_v[s]
        xv = x_v[s]
        vrow = vrow_v[s]
        yv = y_v[s]

        def ebody(e, carry):
            rb = e * L1
            acc = [jnp.zeros((L,), jnp.float32) for _ in range(KD)]
            for l in range(L1):
                for k in range(KD):
                    h = rows[rb + l, pl.ds(k * L, L)]
                    xx = xv[rb + l, pl.ds(k * L, L)]
                    acc[k] = acc[k] + h * xx
            t = jnp.zeros((L,), jnp.float32)
            for k in range(KD):
                w = vrow[e, pl.ds(k * L, L)] * yv[e, pl.ds(k * L, L)]
                t = t + acc[k] * w
            s = jnp.sum(t)
            return jnp.where(lane_iota == e, s, carry)

        tot = lax.fori_loop(0, C, ebody, jnp.zeros((L,), jnp.float32))
        outv[:] = tot
        bb = bbase0 + ch * C
        pltpu.sync_copy(outv, out.at[pl.ds(bb, C)])

    prep_idx(0)
    prep_idx(1)
    prep_main(0)
    for ch in range(NCH):
        if ch + 1 < NCH:
            prep_main(ch + 1)
        compute(ch)
        if ch + 2 < NCH:
            prep_idx(ch + 2)


@jax.jit
def _gmf_sc(xi, yi, xf, yf, human, virus):
    mesh = plsc.VectorSubcoreMesh(core_axis_name="c", subcore_axis_name="s")
    scratch = [
        pltpu.VMEM((RPC,), jnp.int32), pltpu.VMEM((RPC,), jnp.int32),
        pltpu.VMEM((C,), jnp.int32), pltpu.VMEM((C,), jnp.int32),
        pltpu.VMEM((RPC, D), jnp.float32), pltpu.VMEM((RPC, D), jnp.float32),
        pltpu.VMEM((RPC, D), jnp.float32), pltpu.VMEM((RPC, D), jnp.float32),
        pltpu.VMEM((C, D), jnp.float32), pltpu.VMEM((C, D), jnp.float32),
        pltpu.VMEM((C, D), jnp.float32), pltpu.VMEM((C, D), jnp.float32),
        pltpu.VMEM((C,), jnp.float32),
        pltpu.SemaphoreType.DMA, pltpu.SemaphoreType.DMA,
        pltpu.SemaphoreType.DMA, pltpu.SemaphoreType.DMA,
    ]
    run = pl.kernel(
        _body,
        out_type=jax.ShapeDtypeStruct((B,), jnp.float32),
        mesh=mesh,
        scratch_types=scratch,
        compiler_params=pltpu.CompilerParams(
            needs_layout_passes=False, use_tc_tiling_on_sc=False),
    )
    return run(xi, yi, xf, yf, human, virus)


def kernel(x_idx, y_idx, x, y, human_table, virus_table):
    xi = x_idx.reshape(B * L1)
    yi = y_idx.reshape(B)
    xf = x.reshape(B * L1, D)
    out = _gmf_sc(xi, yi, xf, y, human_table, virus_table)
    return out.reshape(B, 1)
